# Initial kernel scaffold; baseline (speedup 1.0000x reference)
import jax
import jax.numpy as jnp
from jax.experimental import pallas as pl


def kernel(keywords, W_proj, b_proj, W_emb):
    raise NotImplementedError("write your pallas kernel here")


# speedup vs baseline: 1.9730x; 1.9730x over previous
import functools

import jax
import jax.numpy as jnp
from jax.experimental import pallas as pl
from jax.experimental.pallas import tpu as pltpu

_EPS = 1e-8
_VB = 2048
_NR = 2


def _fused_kernel(nj, v, kw_ref, wp_ref, b_ref, we_ref,
                  cos_ref, prob_ref, out_ref,
                  kwn_ref, acc_ref, s_ref):
    s = pl.program_id(1)
    j = pl.program_id(2)

    @pl.when(jnp.logical_and(s == 0, j == 0))
    def _init():
        kw = jnp.dot(kw_ref[...], wp_ref[...],
                     preferred_element_type=jnp.float32) + b_ref[...]
        nrm = jnp.sqrt(jnp.sum(kw * kw, axis=1, keepdims=True))
        kwn_ref[...] = kw / jnp.maximum(nrm, _EPS)
        acc_ref[...] = jnp.zeros_like(acc_ref)
        s_ref[...] = jnp.zeros_like(s_ref)

    e = we_ref[...]
    en_nrm = jnp.sqrt(jnp.sum(e * e, axis=1, keepdims=True))
    en = e / jnp.maximum(en_nrm, _EPS)
    kwn = kwn_ref[...]
    cos = jax.lax.dot_general(kwn, en, (((1,), (1,)), ((), ())),
                              preferred_element_type=jnp.float32)

    colid = jax.lax.broadcasted_iota(jnp.int32, (1, _VB), 1) + j * _VB
    p = jnp.where(colid < v, jnp.exp(cos - 1.0), 0.0)

    @pl.when(s == 0)
    def _sweep0():
        cos_ref[...] = cos
        s_ref[...] = s_ref[...] + jnp.sum(p, axis=1, keepdims=True)

    @pl.when(jnp.logical_and(s == 0, j < nj - 1))
    def _acc_full():
        acc_ref[...] = acc_ref[...] + jnp.dot(
            p, e, preferred_element_type=jnp.float32)

    @pl.when(jnp.logical_and(s == 0, j == nj - 1))
    def _acc_ragged():
        rowid = jax.lax.broadcasted_iota(jnp.int32, (_VB, 1), 0) + j * _VB
        e_m = jnp.where(rowid < v, e, 0.0)
        acc_ref[...] = acc_ref[...] + jnp.dot(
            p, e_m, preferred_element_type=jnp.float32)

    @pl.when(s == 1)
    def _sweep1():
        inv = 1.0 / s_ref[...]
        prob_ref[...] = p * inv

        @pl.when(j == 0)
        def _out():
            out_ref[...] = acc_ref[...] * inv


def kernel(keywords, W_proj, b_proj, W_emb):
    bsz, kwn, dm = keywords.shape
    v, td = W_emb.shape
    bk = bsz * kwn
    rb = bk // _NR
    nj = pl.cdiv(v, _VB)

    kw2 = keywords.reshape(bk, dm)
    b2 = b_proj.reshape(1, td)

    cos, prob, out = pl.pallas_call(
        functools.partial(_fused_kernel, nj, v),
        grid=(_NR, 2, nj),
        in_specs=[
            pl.BlockSpec((rb, dm), lambda r, s, j: (r, 0)),
            pl.BlockSpec((dm, td), lambda r, s, j: (0, 0)),
            pl.BlockSpec((1, td), lambda r, s, j: (0, 0)),
            pl.BlockSpec((_VB, td), lambda r, s, j: (j, 0)),
        ],
        out_specs=[
            pl.BlockSpec((rb, _VB),
                         lambda r, s, j: (r, jnp.where(s == 0, j, nj - 1))),
            pl.BlockSpec((rb, _VB),
                         lambda r, s, j: (r, jnp.where(s == 1, j, 0))),
            pl.BlockSpec((rb, td), lambda r, s, j: (r, 0)),
        ],
        out_shape=(
            jax.ShapeDtypeStruct((bk, v), jnp.float32),
            jax.ShapeDtypeStruct((bk, v), jnp.float32),
            jax.ShapeDtypeStruct((bk, td), jnp.float32),
        ),
        scratch_shapes=[
            pltpu.VMEM((rb, td), jnp.float32),
            pltpu.VMEM((rb, td), jnp.float32),
            pltpu.VMEM((rb, 1), jnp.float32),
        ],
        compiler_params=pltpu.CompilerParams(
            dimension_semantics=("parallel", "arbitrary", "arbitrary"),
            vmem_limit_bytes=64 * 1024 * 1024,
        ),
    )(kw2, W_proj, b2, W_emb)

    return (out.reshape(bsz, kwn, td),
            prob.reshape(bsz, kwn, v),
            cos.reshape(bsz, kwn, v))


# speedup vs baseline: 2.0872x; 1.0579x over previous
import functools

import jax
import jax.numpy as jnp
from jax.experimental import pallas as pl
from jax.experimental.pallas import tpu as pltpu

_EPS = 1e-8
_VB = 2048
_SC = 512
_NR = 2


def _fused_kernel(nj, v, kw_ref, wp_ref, b_ref, we_ref,
                  cos_ref, prob_ref, out_ref,
                  kwn_ref, acc_ref, s_ref):
    s = pl.program_id(1)
    j = pl.program_id(2)

    @pl.when(jnp.logical_and(s == 0, j == 0))
    def _init():
        kw = jnp.dot(kw_ref[...], wp_ref[...],
                     preferred_element_type=jnp.float32) + b_ref[...]
        nrm = jnp.sqrt(jnp.sum(kw * kw, axis=1, keepdims=True))
        kwn_ref[...] = (kw / jnp.maximum(nrm, _EPS)).astype(jnp.bfloat16)
        acc_ref[...] = jnp.zeros_like(acc_ref)
        s_ref[...] = jnp.zeros_like(s_ref)

    kwn = kwn_ref[...]
    td = we_ref.shape[1]

    def _sub(c):
        e_c = we_ref[c * _SC:(c + 1) * _SC, :]
        rid = jax.lax.broadcasted_iota(jnp.int32, (_SC, td), 0) + (
            j * _VB + c * _SC)
        e_m = jnp.where(rid < v, e_c, 0.0)
        nrm = jnp.sqrt(jnp.sum(e_m * e_m, axis=1, keepdims=True))
        en = (e_m / jnp.maximum(nrm, _EPS)).astype(jnp.bfloat16)
        cos_c = jax.lax.dot_general(kwn, en, (((1,), (1,)), ((), ())),
                                    preferred_element_type=jnp.float32)
        cid = jax.lax.broadcasted_iota(jnp.int32, (1, _SC), 1) + (
            j * _VB + c * _SC)
        p = jnp.where(cid < v, jnp.exp(cos_c - 1.0), 0.0)
        return e_m, cos_c, p

    @pl.when(s == 0)
    def _sweep0():
        s_loc = s_ref[...]
        for c in range(_VB // _SC):
            e_m, cos_c, p = _sub(c)
            cos_ref[:, c * _SC:(c + 1) * _SC] = cos_c
            s_loc = s_loc + jnp.sum(p, axis=1, keepdims=True)
            acc_ref[...] = acc_ref[...] + jnp.dot(
                p.astype(jnp.bfloat16), e_m.astype(jnp.bfloat16),
                preferred_element_type=jnp.float32)
        s_ref[...] = s_loc

    @pl.when(s == 1)
    def _sweep1():
        inv = 1.0 / s_ref[...]
        for c in range(_VB // _SC):
            _, _, p = _sub(c)
            prob_ref[:, c * _SC:(c + 1) * _SC] = p * inv

        @pl.when(j == 0)
        def _out():
            out_ref[...] = acc_ref[...] * inv


def kernel(keywords, W_proj, b_proj, W_emb):
    bsz, kwn, dm = keywords.shape
    v, td = W_emb.shape
    bk = bsz * kwn
    rb = bk // _NR
    nj = pl.cdiv(v, _VB)

    kw2 = keywords.reshape(bk, dm)
    b2 = b_proj.reshape(1, td)

    cos, prob, out = pl.pallas_call(
        functools.partial(_fused_kernel, nj, v),
        grid=(_NR, 2, nj),
        in_specs=[
            pl.BlockSpec((rb, dm), lambda r, s, j: (r, 0)),
            pl.BlockSpec((dm, td), lambda r, s, j: (0, 0)),
            pl.BlockSpec((1, td), lambda r, s, j: (0, 0)),
            pl.BlockSpec((_VB, td), lambda r, s, j: (j, 0)),
        ],
        out_specs=[
            pl.BlockSpec((rb, _VB),
                         lambda r, s, j: (r, jnp.where(s == 0, j, nj - 1))),
            pl.BlockSpec((rb, _VB),
                         lambda r, s, j: (r, jnp.where(s == 1, j, 0))),
            pl.BlockSpec((rb, td), lambda r, s, j: (r, 0)),
        ],
        out_shape=(
            jax.ShapeDtypeStruct((bk, v), jnp.float32),
            jax.ShapeDtypeStruct((bk, v), jnp.float32),
            jax.ShapeDtypeStruct((bk, td), jnp.float32),
        ),
        scratch_shapes=[
            pltpu.VMEM((rb, td), jnp.bfloat16),
            pltpu.VMEM((rb, td), jnp.float32),
            pltpu.VMEM((rb, 1), jnp.float32),
        ],
        compiler_params=pltpu.CompilerParams(
            dimension_semantics=("parallel", "arbitrary", "arbitrary"),
            vmem_limit_bytes=64 * 1024 * 1024,
        ),
    )(kw2, W_proj, b2, W_emb)

    return (out.reshape(bsz, kwn, td),
            prob.reshape(bsz, kwn, v),
            cos.reshape(bsz, kwn, v))
